# baseline (speedup 1.0000x reference)
import functools
import math

import jax
import jax.numpy as jnp
from jax.experimental import pallas as pl
from jax.experimental.pallas import tpu as pltpu

_B, _N = 4, 256
_F = 64
_H = 64
_ENF = 16
_BI = 128
_NI = _N // _BI
_NH = _N // 2


def _sinusoidal(x, dim):
    half = dim // 2
    freqs = jnp.exp(-jnp.log(10000.0) * jnp.arange(half, dtype=jnp.float32) / half)
    args = x[..., None] * freqs
    return jnp.concatenate([jnp.sin(args), jnp.cos(args)], axis=-1)


def _embed_kernel(feat_ref, w_ref, b_ref, out_ref):
    out_ref[...] = (
        jnp.dot(feat_ref[...], w_ref[...], preferred_element_type=jnp.float32)
        + b_ref[...]
    )


def _fast_sin(x):
    two_pi = 6.283185307179586
    n = jnp.round(x * (1.0 / two_pi))
    y = x - n * two_pi
    y2 = y * y
    p = -2.5052108385441718e-08
    p = p * y2 + 2.7557319223985893e-06
    p = p * y2 - 1.984126984126984e-04
    p = p * y2 + 8.333333333333333e-03
    p = p * y2 - 1.6666666666666666e-01
    return y + y * y2 * p


def _pair_rad(xi, xr):
    return ((xi[:, 0:1] - xr[0:1, :]) ** 2
            + (xi[:, 1:2] - xr[1:2, :]) ** 2
            + (xi[:, 2:3] - xr[2:3, :]) ** 2)


def _layer_kernel(final,
                  xi_ref, xe_ref, xo_ref, xre_ref, xro_ref,
                  bpi_ref, bpre_ref, bpro_ref,
                  hi_ref, hp_ref, na_ref, feat_ref,
                  w1hi_ref, w1hj_ref, w1hj2_ref, weae_ref, weao_ref,
                  wradl_ref, wradr_ref, b1_ref, dvec_ref,
                  w2_ref, w2blk_ref, b2_ref, b22_ref,
                  cw1blk_ref, cb12_ref, cw2t2_ref, cb2_ref,
                  nw1h_ref, nw1a_ref, nw1n_ref, nb1_ref,
                  nw2_ref, nb2_ref,
                  wout_ref, bout_ref,
                  xout_ref, hout_ref):
    f32 = jnp.float32
    xi = xi_ref[0]
    rad_e = _pair_rad(xi, xre_ref[0])
    rad_o = _pair_rad(xi, xro_ref[0])
    norm_e = jnp.sqrt(rad_e + 1e-8)
    norm_o = jnp.sqrt(rad_o + 1e-8)

    bpi = bpi_ref[0]
    d0e = jnp.sqrt(_pair_rad(bpi, bpre_ref[0]) + 1e-8)
    d0o = jnp.sqrt(_pair_rad(bpi, bpro_ref[0]) + 1e-8)

    half = _ENF // 2
    k16 = jax.lax.broadcasted_iota(jnp.int32, (1, 1, _ENF), 2)
    kmod = jnp.bitwise_and(k16, half - 1).astype(f32)
    freq16 = jnp.exp((-math.log(10000.0) / half) * kmod)
    ph16 = jnp.where(k16 >= half, jnp.float32(math.pi / 2), 0.0)
    ea_e = _fast_sin(d0e[:, :, None] * freq16 + ph16)
    ea_o = _fast_sin(d0o[:, :, None] * freq16 + ph16)
    eac = (jnp.dot(ea_e.reshape(_BI * _NH, _ENF).astype(jnp.bfloat16),
                   weae_ref[...], preferred_element_type=f32)
           + jnp.dot(ea_o.reshape(_BI * _NH, _ENF).astype(jnp.bfloat16),
                     weao_ref[...], preferred_element_type=f32))

    hi = hi_ref[0]
    hp = hp_ref[0]
    a_i = jnp.dot(hi, w1hi_ref[...], preferred_element_type=f32) + b1_ref[...]
    a_i2 = jnp.concatenate([a_i, a_i], axis=1)
    a_j2 = jnp.dot(hp, w1hj2_ref[...], preferred_element_type=f32)

    m1 = jax.nn.silu(a_i2[:, None, :] + a_j2[None, :, :]
                     + eac.reshape(_BI, _NH, 2 * _H)
                     + rad_e[:, :, None] * wradl_ref[...].reshape(1, 1, 2 * _H)
                     + rad_o[:, :, None] * wradr_ref[...].reshape(1, 1, 2 * _H))
    m2 = jax.nn.silu(
        jnp.dot(m1.reshape(_BI * _NH, 2 * _H).astype(jnp.bfloat16),
                w2blk_ref[...],
                preferred_element_type=f32) + b22_ref[...])
    aggp = jnp.sum(m2.reshape(_BI, _NH, 2 * _H), axis=1)

    a_hj = jnp.dot(hi, w1hj_ref[...], preferred_element_type=f32)
    m1d = jax.nn.silu(a_i + a_hj + dvec_ref[...])
    m2d = jax.nn.silu(jnp.dot(m1d, w2_ref[...], preferred_element_type=f32)
                      + b2_ref[...])
    agg = aggp[:, :_H] + aggp[:, _H:] - m2d

    cwa = jax.nn.silu(
        jnp.dot(m2.astype(jnp.bfloat16), cw1blk_ref[...],
                preferred_element_type=f32)
        + cb12_ref[...])
    cwp = cwa.reshape(_BI, _NH, 2 * _H) * cw2t2_ref[...].reshape(1, 1, 2 * _H)
    cw_e = jnp.sum(cwp[:, :, :_H], axis=2) + cb2_ref[...]
    cw_o = jnp.sum(cwp[:, :, _H:], axis=2) + cb2_ref[...]

    gi = (jax.lax.broadcasted_iota(jnp.int32, (_BI, _NH), 0)
          + pl.program_id(1) * _BI)
    jj = jax.lax.broadcasted_iota(jnp.int32, (_BI, _NH), 1)
    adj_e = jnp.where(2 * jj == gi, 0.0, 1.0)
    adj_o = jnp.where(2 * jj + 1 == gi, 0.0, 1.0)
    wc_e = cw_e * adj_e / (norm_e + 1.0)
    wc_o = cw_o * adj_o / (norm_o + 1.0)
    rowsum = (jnp.sum(wc_e, axis=1, keepdims=True)
              + jnp.sum(wc_o, axis=1, keepdims=True))
    sj = (jnp.dot(wc_e, xe_ref[0], preferred_element_type=f32)
          + jnp.dot(wc_o, xo_ref[0], preferred_element_type=f32))
    x_new = xi + (xi * rowsum - sj) * (1.0 / (_N - 1))

    out = jax.nn.silu(
        jnp.dot(hi, nw1h_ref[...], preferred_element_type=f32)
        + jnp.dot(agg, nw1a_ref[...], preferred_element_type=f32)
        + jnp.dot(na_ref[0], nw1n_ref[...], preferred_element_type=f32)
        + nb1_ref[...])
    out = jnp.dot(out, nw2_ref[...], preferred_element_type=f32) + nb2_ref[...]
    h_new = hi + out

    if final:
        xout_ref[0] = x_new - bpi
        hout_ref[0] = (jnp.dot(h_new, wout_ref[...],
                               preferred_element_type=f32)
                       + bout_ref[...]) - feat_ref[0]
    else:
        xout_ref[0] = x_new
        hout_ref[0] = h_new


def _blockdiag(w):
    z = jnp.zeros_like(w)
    return jnp.block([[w, z], [z, w]])


def _egnn_layer(x, bb_parts, h, node_attr, features, lp, wout, bout, final):
    bpi_a, bpre_a, bpro_a = bb_parts
    xe = x[:, 0::2, :]
    xo = x[:, 1::2, :]
    xre = jnp.swapaxes(xe, 1, 2)
    xro = jnp.swapaxes(xo, 1, 2)
    h_pair = h.reshape(_B, _NH, 2 * _H)

    ew1 = lp["edge_w1"]
    w1hi, w1hj = ew1[0:_H], ew1[_H:2 * _H]
    w1rad, w1ea = ew1[2 * _H:2 * _H + 1], ew1[2 * _H + 1:]
    zea = jnp.zeros_like(w1ea)
    weae = jnp.concatenate([w1ea, zea], axis=1).astype(jnp.bfloat16)
    weao = jnp.concatenate([zea, w1ea], axis=1).astype(jnp.bfloat16)
    zrad = jnp.zeros_like(w1rad)
    wradl = jnp.concatenate([w1rad, zrad], axis=1)
    wradr = jnp.concatenate([zrad, w1rad], axis=1)
    w1hj2 = _blockdiag(w1hj)
    w2blk = _blockdiag(lp["edge_w2"])
    cw1blk = _blockdiag(lp["coord_w1"])
    d_diag = jnp.float32(math.sqrt(1e-8))
    ea_diag = _sinusoidal(d_diag[None], _ENF)
    dvec = ea_diag @ w1ea

    nw1 = lp["node_w1"]
    nw1h, nw1a, nw1n = nw1[0:_H], nw1[_H:2 * _H], nw1[2 * _H:]

    def r2(v):
        return v.reshape(1, -1)

    b2r = r2(lp["edge_b2"])
    cb1r = r2(lp["coord_b1"])

    blk_i3 = pl.BlockSpec((1, _BI, 3), lambda b, i: (b, i, 0))
    blk_h3 = pl.BlockSpec((1, _NH, 3), lambda b, i: (b, 0, 0))
    blk_3h = pl.BlockSpec((1, 3, _NH), lambda b, i: (b, 0, 0))
    blk_ih = pl.BlockSpec((1, _BI, _H), lambda b, i: (b, i, 0))
    blk_p = pl.BlockSpec((1, _NH, 2 * _H), lambda b, i: (b, 0, 0))

    def wspec(a):
        return pl.BlockSpec(a.shape, lambda b, i: tuple(0 for _ in a.shape))

    weights = [w1hi, w1hj, w1hj2, weae, weao, wradl, wradr, r2(lp["edge_b1"]),
               dvec,
               lp["edge_w2"], w2blk.astype(jnp.bfloat16), b2r,
               jnp.concatenate([b2r, b2r], axis=1),
               cw1blk.astype(jnp.bfloat16),
               jnp.concatenate([cb1r, cb1r], axis=1),
               jnp.concatenate([lp["coord_w2"].reshape(1, _H)] * 2, axis=1),
               lp["coord_b2"].reshape(1, 1),
               nw1h, nw1a, nw1n, r2(lp["node_b1"]),
               lp["node_w2"], r2(lp["node_b2"]),
               wout, r2(bout)]

    return pl.pallas_call(
        functools.partial(_layer_kernel, final),
        grid=(_B, _NI),
        in_specs=[blk_i3, blk_h3, blk_h3, blk_3h, blk_3h,
                  blk_i3, blk_3h, blk_3h,
                  blk_ih, blk_p, blk_ih, blk_ih]
                 + [wspec(w) for w in weights],
        out_specs=[blk_i3, blk_ih],
        out_shape=[jax.ShapeDtypeStruct((_B, _N, 3), jnp.float32),
                   jax.ShapeDtypeStruct((_B, _N, _H), jnp.float32)],
        compiler_params=pltpu.CompilerParams(
            dimension_semantics=("parallel", "parallel")),
    )(x, xe, xo, xre, xro, bpi_a, bpre_a, bpro_a,
      h, h_pair, node_attr, features, *weights)


def kernel(coordinates, features, idx, params):
    bb_pos = coordinates.astype(jnp.float32)
    bb_feat = features.astype(jnp.float32)
    bpe = bb_pos[:, 0::2, :]
    bpo = bb_pos[:, 1::2, :]
    bb_parts = (bb_pos, jnp.swapaxes(bpe, 1, 2), jnp.swapaxes(bpo, 1, 2))

    pos_ids = jnp.arange(_N, dtype=jnp.float32)
    embed_N = _sinusoidal(pos_ids, _H)
    embed_T = _sinusoidal(idx.astype(jnp.float32), _H)
    node_attr = (embed_N[None, :, :] + embed_T[:, None, :]).astype(jnp.float32)

    win, bin_ = params["emb_in"]
    h0 = pl.pallas_call(
        _embed_kernel,
        out_shape=jax.ShapeDtypeStruct((_B * _N, _H), jnp.float32),
    )(bb_feat.reshape(_B * _N, _F), win, bin_.reshape(1, _H))
    h = h0.reshape(_B, _N, _H)

    wout, bout = params["emb_out"]
    x = bb_pos
    n_layers = len(params["layers"])
    for li, lp in enumerate(params["layers"]):
        final = li == n_layers - 1
        x, h = _egnn_layer(x, bb_parts, h, node_attr, bb_feat, lp,
                           wout, bout, final)

    return (x, h)


# speedup vs baseline: 1.2848x; 1.2848x over previous
import functools
import math

import jax
import jax.numpy as jnp
from jax.experimental import pallas as pl
from jax.experimental.pallas import tpu as pltpu

_B, _N = 4, 256
_F = 64
_H = 64
_ENF = 16
_BI = 64
_NI = _N // _BI
_NH = _N // 2


def _sinusoidal(x, dim):
    half = dim // 2
    freqs = jnp.exp(-jnp.log(10000.0) * jnp.arange(half, dtype=jnp.float32) / half)
    args = x[..., None] * freqs
    return jnp.concatenate([jnp.sin(args), jnp.cos(args)], axis=-1)


def _embed_kernel(feat_ref, w_ref, b_ref, out_ref):
    out_ref[...] = (
        jnp.dot(feat_ref[...], w_ref[...], preferred_element_type=jnp.float32)
        + b_ref[...]
    )


def _fast_sin(x):
    two_pi = 6.283185307179586
    magic = 12582912.0
    n = (x * (1.0 / two_pi) + magic) - magic
    y = x - n * two_pi
    y2 = y * y
    p = -2.5052108385441718e-08
    p = p * y2 + 2.7557319223985893e-06
    p = p * y2 - 1.984126984126984e-04
    p = p * y2 + 8.333333333333333e-03
    p = p * y2 - 1.6666666666666666e-01
    return y + y * y2 * p


def _pair_rad(xi, xr):
    return ((xi[:, 0:1] - xr[0:1, :]) ** 2
            + (xi[:, 1:2] - xr[1:2, :]) ** 2
            + (xi[:, 2:3] - xr[2:3, :]) ** 2)


def _layer_kernel(final,
                  xi_ref, xe_ref, xo_ref, xre_ref, xro_ref,
                  bpi_ref, bpre_ref, bpro_ref,
                  hi_ref, hp_ref, na_ref, feat_ref,
                  w1hi_ref, w1hj_ref, w1hj2_ref, weae_ref, weao_ref,
                  wradl_ref, wradr_ref, b1_ref, dvec_ref,
                  w2_ref, w2blk_ref, b2_ref, b22_ref,
                  cw1blk_ref, cb12_ref, cw2t2_ref, cb2_ref,
                  nw1h_ref, nw1a_ref, nw1n_ref, nb1_ref,
                  nw2_ref, nb2_ref,
                  wout_ref, bout_ref,
                  xout_ref, hout_ref):
    f32 = jnp.float32
    xi = xi_ref[0]
    rad_e = _pair_rad(xi, xre_ref[0])
    rad_o = _pair_rad(xi, xro_ref[0])
    norm_e = jnp.sqrt(rad_e + 1e-8)
    norm_o = jnp.sqrt(rad_o + 1e-8)

    bpi = bpi_ref[0]
    d0e = jnp.sqrt(_pair_rad(bpi, bpre_ref[0]) + 1e-8)
    d0o = jnp.sqrt(_pair_rad(bpi, bpro_ref[0]) + 1e-8)

    half = _ENF // 2
    k16 = jax.lax.broadcasted_iota(jnp.int32, (1, 1, _ENF), 2)
    kmod = jnp.bitwise_and(k16, half - 1).astype(f32)
    freq16 = jnp.exp((-math.log(10000.0) / half) * kmod)
    ph16 = jnp.where(k16 >= half, jnp.float32(math.pi / 2), 0.0)
    ea_e = _fast_sin(d0e[:, :, None] * freq16 + ph16)
    ea_o = _fast_sin(d0o[:, :, None] * freq16 + ph16)
    eac = (jnp.dot(ea_e.reshape(_BI * _NH, _ENF).astype(jnp.bfloat16),
                   weae_ref[...], preferred_element_type=f32)
           + jnp.dot(ea_o.reshape(_BI * _NH, _ENF).astype(jnp.bfloat16),
                     weao_ref[...], preferred_element_type=f32))

    hi = hi_ref[0]
    hp = hp_ref[0]
    a_i = jnp.dot(hi, w1hi_ref[...], preferred_element_type=f32) + b1_ref[...]
    a_i2 = jnp.concatenate([a_i, a_i], axis=1)
    a_j2 = jnp.dot(hp, w1hj2_ref[...], preferred_element_type=f32)

    m1 = jax.nn.silu(a_i2[:, None, :] + a_j2[None, :, :]
                     + eac.reshape(_BI, _NH, 2 * _H)
                     + rad_e[:, :, None] * wradl_ref[...].reshape(1, 1, 2 * _H)
                     + rad_o[:, :, None] * wradr_ref[...].reshape(1, 1, 2 * _H))
    m2 = jax.nn.silu(
        jnp.dot(m1.reshape(_BI * _NH, 2 * _H).astype(jnp.bfloat16),
                w2blk_ref[...],
                preferred_element_type=f32) + b22_ref[...])
    aggp = jnp.sum(m2.reshape(_BI, _NH, 2 * _H), axis=1)

    a_hj = jnp.dot(hi, w1hj_ref[...], preferred_element_type=f32)
    m1d = jax.nn.silu(a_i + a_hj + dvec_ref[...])
    m2d = jax.nn.silu(jnp.dot(m1d, w2_ref[...], preferred_element_type=f32)
                      + b2_ref[...])
    agg = aggp[:, :_H] + aggp[:, _H:] - m2d

    cwa = jax.nn.silu(
        jnp.dot(m2.astype(jnp.bfloat16), cw1blk_ref[...],
                preferred_element_type=f32)
        + cb12_ref[...])
    cwp = cwa.reshape(_BI, _NH, 2 * _H) * cw2t2_ref[...].reshape(1, 1, 2 * _H)
    cw_e = jnp.sum(cwp[:, :, :_H], axis=2) + cb2_ref[...]
    cw_o = jnp.sum(cwp[:, :, _H:], axis=2) + cb2_ref[...]

    gi = (jax.lax.broadcasted_iota(jnp.int32, (_BI, _NH), 0)
          + pl.program_id(1) * _BI)
    jj = jax.lax.broadcasted_iota(jnp.int32, (_BI, _NH), 1)
    adj_e = jnp.where(2 * jj == gi, 0.0, 1.0)
    adj_o = jnp.where(2 * jj + 1 == gi, 0.0, 1.0)
    re_ = jax.lax.rsqrt((norm_e + 1.0) * (norm_e + 1.0))
    ro_ = jax.lax.rsqrt((norm_o + 1.0) * (norm_o + 1.0))
    wc_e = cw_e * adj_e * (re_ * re_)
    wc_o = cw_o * adj_o * (ro_ * ro_)
    rowsum = (jnp.sum(wc_e, axis=1, keepdims=True)
              + jnp.sum(wc_o, axis=1, keepdims=True))
    sj = (jnp.dot(wc_e, xe_ref[0], preferred_element_type=f32)
          + jnp.dot(wc_o, xo_ref[0], preferred_element_type=f32))
    x_new = xi + (xi * rowsum - sj) * (1.0 / (_N - 1))

    out = jax.nn.silu(
        jnp.dot(hi, nw1h_ref[...], preferred_element_type=f32)
        + jnp.dot(agg, nw1a_ref[...], preferred_element_type=f32)
        + jnp.dot(na_ref[0], nw1n_ref[...], preferred_element_type=f32)
        + nb1_ref[...])
    out = jnp.dot(out, nw2_ref[...], preferred_element_type=f32) + nb2_ref[...]
    h_new = hi + out

    if final:
        xout_ref[0] = x_new - bpi
        hout_ref[0] = (jnp.dot(h_new, wout_ref[...],
                               preferred_element_type=f32)
                       + bout_ref[...]) - feat_ref[0]
    else:
        xout_ref[0] = x_new
        hout_ref[0] = h_new


def _blockdiag(w):
    z = jnp.zeros_like(w)
    return jnp.block([[w, z], [z, w]])


def _egnn_layer(x, bb_parts, h, node_attr, features, lp, wout, bout, final):
    bpi_a, bpre_a, bpro_a = bb_parts
    xe = x[:, 0::2, :]
    xo = x[:, 1::2, :]
    xre = jnp.swapaxes(xe, 1, 2)
    xro = jnp.swapaxes(xo, 1, 2)
    h_pair = h.reshape(_B, _NH, 2 * _H)

    ew1 = lp["edge_w1"]
    w1hi, w1hj = ew1[0:_H], ew1[_H:2 * _H]
    w1rad, w1ea = ew1[2 * _H:2 * _H + 1], ew1[2 * _H + 1:]
    zea = jnp.zeros_like(w1ea)
    weae = jnp.concatenate([w1ea, zea], axis=1).astype(jnp.bfloat16)
    weao = jnp.concatenate([zea, w1ea], axis=1).astype(jnp.bfloat16)
    zrad = jnp.zeros_like(w1rad)
    wradl = jnp.concatenate([w1rad, zrad], axis=1)
    wradr = jnp.concatenate([zrad, w1rad], axis=1)
    w1hj2 = _blockdiag(w1hj)
    w2blk = _blockdiag(lp["edge_w2"])
    cw1blk = _blockdiag(lp["coord_w1"])
    d_diag = jnp.float32(math.sqrt(1e-8))
    ea_diag = _sinusoidal(d_diag[None], _ENF)
    dvec = ea_diag @ w1ea

    nw1 = lp["node_w1"]
    nw1h, nw1a, nw1n = nw1[0:_H], nw1[_H:2 * _H], nw1[2 * _H:]

    def r2(v):
        return v.reshape(1, -1)

    b2r = r2(lp["edge_b2"])
    cb1r = r2(lp["coord_b1"])

    blk_i3 = pl.BlockSpec((1, _BI, 3), lambda b, i: (b, i, 0))
    blk_h3 = pl.BlockSpec((1, _NH, 3), lambda b, i: (b, 0, 0))
    blk_3h = pl.BlockSpec((1, 3, _NH), lambda b, i: (b, 0, 0))
    blk_ih = pl.BlockSpec((1, _BI, _H), lambda b, i: (b, i, 0))
    blk_p = pl.BlockSpec((1, _NH, 2 * _H), lambda b, i: (b, 0, 0))

    def wspec(a):
        return pl.BlockSpec(a.shape, lambda b, i: tuple(0 for _ in a.shape))

    weights = [w1hi, w1hj, w1hj2, weae, weao, wradl, wradr, r2(lp["edge_b1"]),
               dvec,
               lp["edge_w2"], w2blk.astype(jnp.bfloat16), b2r,
               jnp.concatenate([b2r, b2r], axis=1),
               cw1blk.astype(jnp.bfloat16),
               jnp.concatenate([cb1r, cb1r], axis=1),
               jnp.concatenate([lp["coord_w2"].reshape(1, _H)] * 2, axis=1),
               lp["coord_b2"].reshape(1, 1),
               nw1h, nw1a, nw1n, r2(lp["node_b1"]),
               lp["node_w2"], r2(lp["node_b2"]),
               wout, r2(bout)]

    return pl.pallas_call(
        functools.partial(_layer_kernel, final),
        grid=(_B, _NI),
        in_specs=[blk_i3, blk_h3, blk_h3, blk_3h, blk_3h,
                  blk_i3, blk_3h, blk_3h,
                  blk_ih, blk_p, blk_ih, blk_ih]
                 + [wspec(w) for w in weights],
        out_specs=[blk_i3, blk_ih],
        out_shape=[jax.ShapeDtypeStruct((_B, _N, 3), jnp.float32),
                   jax.ShapeDtypeStruct((_B, _N, _H), jnp.float32)],
        compiler_params=pltpu.CompilerParams(
            dimension_semantics=("parallel", "parallel")),
    )(x, xe, xo, xre, xro, bpi_a, bpre_a, bpro_a,
      h, h_pair, node_attr, features, *weights)


def kernel(coordinates, features, idx, params):
    bb_pos = coordinates.astype(jnp.float32)
    bb_feat = features.astype(jnp.float32)
    bpe = bb_pos[:, 0::2, :]
    bpo = bb_pos[:, 1::2, :]
    bb_parts = (bb_pos, jnp.swapaxes(bpe, 1, 2), jnp.swapaxes(bpo, 1, 2))

    pos_ids = jnp.arange(_N, dtype=jnp.float32)
    embed_N = _sinusoidal(pos_ids, _H)
    embed_T = _sinusoidal(idx.astype(jnp.float32), _H)
    node_attr = (embed_N[None, :, :] + embed_T[:, None, :]).astype(jnp.float32)

    win, bin_ = params["emb_in"]
    h0 = pl.pallas_call(
        _embed_kernel,
        out_shape=jax.ShapeDtypeStruct((_B * _N, _H), jnp.float32),
    )(bb_feat.reshape(_B * _N, _F), win, bin_.reshape(1, _H))
    h = h0.reshape(_B, _N, _H)

    wout, bout = params["emb_out"]
    x = bb_pos
    n_layers = len(params["layers"])
    for li, lp in enumerate(params["layers"]):
        final = li == n_layers - 1
        x, h = _egnn_layer(x, bb_parts, h, node_attr, bb_feat, lp,
                           wout, bout, final)

    return (x, h)


# speedup vs baseline: 1.2850x; 1.0001x over previous
import functools
import math

import jax
import jax.numpy as jnp
from jax.experimental import pallas as pl
from jax.experimental.pallas import tpu as pltpu

_B, _N = 4, 256
_F = 64
_H = 64
_ENF = 16
_BI = 64
_NI = _N // _BI
_NH = _N // 2


def _sinusoidal(x, dim):
    half = dim // 2
    freqs = jnp.exp(-jnp.log(10000.0) * jnp.arange(half, dtype=jnp.float32) / half)
    args = x[..., None] * freqs
    return jnp.concatenate([jnp.sin(args), jnp.cos(args)], axis=-1)


def _embed_kernel(feat_ref, w_ref, b_ref, out_ref):
    out_ref[...] = (
        jnp.dot(feat_ref[...], w_ref[...], preferred_element_type=jnp.float32)
        + b_ref[...]
    )


def _fast_sin(x):
    two_pi = 6.283185307179586
    magic = 12582912.0
    n = (x * (1.0 / two_pi) + magic) - magic
    y = x - n * two_pi
    y2 = y * y
    p = -2.5052108385441718e-08
    p = p * y2 + 2.7557319223985893e-06
    p = p * y2 - 1.984126984126984e-04
    p = p * y2 + 8.333333333333333e-03
    p = p * y2 - 1.6666666666666666e-01
    return y + y * y2 * p


def _pair_rad(xi, xr):
    return ((xi[:, 0:1] - xr[0:1, :]) ** 2
            + (xi[:, 1:2] - xr[1:2, :]) ** 2
            + (xi[:, 2:3] - xr[2:3, :]) ** 2)


def _layer_kernel(final,
                  xi_ref, xe_ref, xo_ref, xre_ref, xro_ref,
                  bpi_ref, bpre_ref, bpro_ref,
                  hi_ref, hp_ref, na_ref, feat_ref,
                  w1hi_ref, w1hj_ref, w1hj2_ref, weae_ref, weao_ref,
                  wradl_ref, wradr_ref, b1_ref, dvec_ref,
                  w2_ref, w2blk_ref, b2_ref, b22_ref,
                  cw1blk_ref, cb12_ref, cw2t2_ref, cb2_ref,
                  nw1h_ref, nw1a_ref, nw1n_ref, nb1_ref,
                  nw2_ref, nb2_ref,
                  wout_ref, bout_ref,
                  xout_ref, hout_ref):
    f32 = jnp.float32
    xi = xi_ref[0]
    rad_e = _pair_rad(xi, xre_ref[0])
    rad_o = _pair_rad(xi, xro_ref[0])
    norm_e = jnp.sqrt(rad_e + 1e-8)
    norm_o = jnp.sqrt(rad_o + 1e-8)

    bpi = bpi_ref[0]
    d0e = jnp.sqrt(_pair_rad(bpi, bpre_ref[0]) + 1e-8)
    d0o = jnp.sqrt(_pair_rad(bpi, bpro_ref[0]) + 1e-8)

    half = _ENF // 2
    k16 = jax.lax.broadcasted_iota(jnp.int32, (1, 1, _ENF), 2)
    kmod = jnp.bitwise_and(k16, half - 1).astype(f32)
    freq16 = jnp.exp((-math.log(10000.0) / half) * kmod)
    ph16 = jnp.where(k16 >= half, jnp.float32(math.pi / 2), 0.0)
    ea_e = _fast_sin(d0e[:, :, None] * freq16 + ph16)
    ea_o = _fast_sin(d0o[:, :, None] * freq16 + ph16)
    eac = (jnp.dot(ea_e.reshape(_BI * _NH, _ENF).astype(jnp.bfloat16),
                   weae_ref[...], preferred_element_type=f32)
           + jnp.dot(ea_o.reshape(_BI * _NH, _ENF).astype(jnp.bfloat16),
                     weao_ref[...], preferred_element_type=f32))

    hi = hi_ref[0]
    hp = hp_ref[0]
    a_i = jnp.dot(hi, w1hi_ref[...], preferred_element_type=f32) + b1_ref[...]
    a_i2 = jnp.concatenate([a_i, a_i], axis=1)
    a_j2 = jnp.dot(hp, w1hj2_ref[...], preferred_element_type=f32)

    m1 = jax.nn.silu(a_i2[:, None, :] + a_j2[None, :, :]
                     + eac.reshape(_BI, _NH, 2 * _H)
                     + rad_e[:, :, None] * wradl_ref[...].reshape(1, 1, 2 * _H)
                     + rad_o[:, :, None] * wradr_ref[...].reshape(1, 1, 2 * _H))
    m2 = jax.nn.silu(
        jnp.dot(m1.reshape(_BI * _NH, 2 * _H).astype(jnp.bfloat16),
                w2blk_ref[...],
                preferred_element_type=f32) + b22_ref[...])
    aggp = jnp.sum(m2.reshape(_BI, _NH, 2 * _H), axis=1)

    a_hj = jnp.dot(hi, w1hj_ref[...], preferred_element_type=f32)
    m1d = jax.nn.silu(a_i + a_hj + dvec_ref[...])
    m2d = jax.nn.silu(jnp.dot(m1d, w2_ref[...], preferred_element_type=f32)
                      + b2_ref[...])
    agg = aggp[:, :_H] + aggp[:, _H:] - m2d

    cwa = jax.nn.silu(
        jnp.dot(m2.astype(jnp.bfloat16), cw1blk_ref[...],
                preferred_element_type=f32)
        + cb12_ref[...])
    cwp = cwa.reshape(_BI, _NH, 2 * _H) * cw2t2_ref[...].reshape(1, 1, 2 * _H)
    cw_e = jnp.sum(cwp[:, :, :_H], axis=2) + cb2_ref[...]
    cw_o = jnp.sum(cwp[:, :, _H:], axis=2) + cb2_ref[...]

    gi = (jax.lax.broadcasted_iota(jnp.int32, (_BI, _NH), 0)
          + pl.program_id(1) * _BI)
    jj = jax.lax.broadcasted_iota(jnp.int32, (_BI, _NH), 1)
    adj_e = jnp.where(2 * jj == gi, 0.0, 1.0)
    adj_o = jnp.where(2 * jj + 1 == gi, 0.0, 1.0)
    re_ = jax.lax.rsqrt((norm_e + 1.0) * (norm_e + 1.0))
    ro_ = jax.lax.rsqrt((norm_o + 1.0) * (norm_o + 1.0))
    wc_e = cw_e * adj_e * re_
    wc_o = cw_o * adj_o * ro_
    rowsum = (jnp.sum(wc_e, axis=1, keepdims=True)
              + jnp.sum(wc_o, axis=1, keepdims=True))
    sj = (jnp.dot(wc_e, xe_ref[0], preferred_element_type=f32)
          + jnp.dot(wc_o, xo_ref[0], preferred_element_type=f32))
    x_new = xi + (xi * rowsum - sj) * (1.0 / (_N - 1))

    out = jax.nn.silu(
        jnp.dot(hi, nw1h_ref[...], preferred_element_type=f32)
        + jnp.dot(agg, nw1a_ref[...], preferred_element_type=f32)
        + jnp.dot(na_ref[0], nw1n_ref[...], preferred_element_type=f32)
        + nb1_ref[...])
    out = jnp.dot(out, nw2_ref[...], preferred_element_type=f32) + nb2_ref[...]
    h_new = hi + out

    if final:
        xout_ref[0] = x_new - bpi
        hout_ref[0] = (jnp.dot(h_new, wout_ref[...],
                               preferred_element_type=f32)
                       + bout_ref[...]) - feat_ref[0]
    else:
        xout_ref[0] = x_new
        hout_ref[0] = h_new


def _blockdiag(w):
    z = jnp.zeros_like(w)
    return jnp.block([[w, z], [z, w]])


def _egnn_layer(x, bb_parts, h, node_attr, features, lp, wout, bout, final):
    bpi_a, bpre_a, bpro_a = bb_parts
    xe = x[:, 0::2, :]
    xo = x[:, 1::2, :]
    xre = jnp.swapaxes(xe, 1, 2)
    xro = jnp.swapaxes(xo, 1, 2)
    h_pair = h.reshape(_B, _NH, 2 * _H)

    ew1 = lp["edge_w1"]
    w1hi, w1hj = ew1[0:_H], ew1[_H:2 * _H]
    w1rad, w1ea = ew1[2 * _H:2 * _H + 1], ew1[2 * _H + 1:]
    zea = jnp.zeros_like(w1ea)
    weae = jnp.concatenate([w1ea, zea], axis=1).astype(jnp.bfloat16)
    weao = jnp.concatenate([zea, w1ea], axis=1).astype(jnp.bfloat16)
    zrad = jnp.zeros_like(w1rad)
    wradl = jnp.concatenate([w1rad, zrad], axis=1)
    wradr = jnp.concatenate([zrad, w1rad], axis=1)
    w1hj2 = _blockdiag(w1hj)
    w2blk = _blockdiag(lp["edge_w2"])
    cw1blk = _blockdiag(lp["coord_w1"])
    d_diag = jnp.float32(math.sqrt(1e-8))
    ea_diag = _sinusoidal(d_diag[None], _ENF)
    dvec = ea_diag @ w1ea

    nw1 = lp["node_w1"]
    nw1h, nw1a, nw1n = nw1[0:_H], nw1[_H:2 * _H], nw1[2 * _H:]

    def r2(v):
        return v.reshape(1, -1)

    b2r = r2(lp["edge_b2"])
    cb1r = r2(lp["coord_b1"])

    blk_i3 = pl.BlockSpec((1, _BI, 3), lambda b, i: (b, i, 0))
    blk_h3 = pl.BlockSpec((1, _NH, 3), lambda b, i: (b, 0, 0))
    blk_3h = pl.BlockSpec((1, 3, _NH), lambda b, i: (b, 0, 0))
    blk_ih = pl.BlockSpec((1, _BI, _H), lambda b, i: (b, i, 0))
    blk_p = pl.BlockSpec((1, _NH, 2 * _H), lambda b, i: (b, 0, 0))

    def wspec(a):
        return pl.BlockSpec(a.shape, lambda b, i: tuple(0 for _ in a.shape))

    weights = [w1hi, w1hj, w1hj2, weae, weao, wradl, wradr, r2(lp["edge_b1"]),
               dvec,
               lp["edge_w2"], w2blk.astype(jnp.bfloat16), b2r,
               jnp.concatenate([b2r, b2r], axis=1),
               cw1blk.astype(jnp.bfloat16),
               jnp.concatenate([cb1r, cb1r], axis=1),
               jnp.concatenate([lp["coord_w2"].reshape(1, _H)] * 2, axis=1),
               lp["coord_b2"].reshape(1, 1),
               nw1h, nw1a, nw1n, r2(lp["node_b1"]),
               lp["node_w2"], r2(lp["node_b2"]),
               wout, r2(bout)]

    return pl.pallas_call(
        functools.partial(_layer_kernel, final),
        grid=(_B, _NI),
        in_specs=[blk_i3, blk_h3, blk_h3, blk_3h, blk_3h,
                  blk_i3, blk_3h, blk_3h,
                  blk_ih, blk_p, blk_ih, blk_ih]
                 + [wspec(w) for w in weights],
        out_specs=[blk_i3, blk_ih],
        out_shape=[jax.ShapeDtypeStruct((_B, _N, 3), jnp.float32),
                   jax.ShapeDtypeStruct((_B, _N, _H), jnp.float32)],
        compiler_params=pltpu.CompilerParams(
            dimension_semantics=("parallel", "parallel")),
    )(x, xe, xo, xre, xro, bpi_a, bpre_a, bpro_a,
      h, h_pair, node_attr, features, *weights)


def kernel(coordinates, features, idx, params):
    bb_pos = coordinates.astype(jnp.float32)
    bb_feat = features.astype(jnp.float32)
    bpe = bb_pos[:, 0::2, :]
    bpo = bb_pos[:, 1::2, :]
    bb_parts = (bb_pos, jnp.swapaxes(bpe, 1, 2), jnp.swapaxes(bpo, 1, 2))

    pos_ids = jnp.arange(_N, dtype=jnp.float32)
    embed_N = _sinusoidal(pos_ids, _H)
    embed_T = _sinusoidal(idx.astype(jnp.float32), _H)
    node_attr = (embed_N[None, :, :] + embed_T[:, None, :]).astype(jnp.float32)

    win, bin_ = params["emb_in"]
    h0 = pl.pallas_call(
        _embed_kernel,
        out_shape=jax.ShapeDtypeStruct((_B * _N, _H), jnp.float32),
    )(bb_feat.reshape(_B * _N, _F), win, bin_.reshape(1, _H))
    h = h0.reshape(_B, _N, _H)

    wout, bout = params["emb_out"]
    x = bb_pos
    n_layers = len(params["layers"])
    for li, lp in enumerate(params["layers"]):
        final = li == n_layers - 1
        x, h = _egnn_layer(x, bb_parts, h, node_attr, bb_feat, lp,
                           wout, bout, final)

    return (x, h)
